# baseline (speedup 1.0000x reference)
import functools

import jax
import jax.numpy as jnp
from jax import lax
from jax.experimental import pallas as pl
from jax.experimental.pallas import tpu as pltpu


def _pack_bf16_pair(hi_f32, lo_f32):
    hi = pltpu.bitcast(hi_f32.astype(jnp.bfloat16).astype(jnp.float32),
                       jnp.uint32)
    lo = pltpu.bitcast(lo_f32.astype(jnp.bfloat16).astype(jnp.float32),
                       jnp.uint32)
    return hi | (lo >> 16)


def _tables_kernel(emb0_ref, emb1_ref, inter_l_ref, inter_r_ref,
                   w00_ref, w01_ref, w10_ref, w11_ref,
                   rw_l_ref, rw_r_ref, rb_l_ref, rb_r_ref,
                   tl_ref, *, n_steps, span):
    s = pl.program_id(0)
    zmid = (n_steps - 1) // 2
    eff = s - jnp.where(s > zmid, 1, 0)

    @pl.when(jnp.logical_and(s != zmid, eff % 2 == 0))
    def _t0():
        h0 = jnp.tanh(lax.dot_general(emb0_ref[...], w00_ref[...],
                                      (((1,), (1,)), ((), ())),
                                      preferred_element_type=jnp.float32))
        t0 = lax.dot_general(h0, w01_ref[...], (((1,), (1,)), ((), ())),
                             preferred_element_type=jnp.float32)
        tl_ref[...] = _pack_bf16_pair(t0, jnp.zeros_like(t0))

    @pl.when(jnp.logical_and(s != zmid, eff % 2 == 1))
    def _t1():
        h1 = jnp.tanh(lax.dot_general(emb1_ref[...], w10_ref[...],
                                      (((1,), (1,)), ((), ())),
                                      preferred_element_type=jnp.float32))
        t1 = lax.dot_general(h1, w11_ref[...], (((1,), (1,)), ((), ())),
                             preferred_element_type=jnp.float32)
        f = jnp.tanh(t1)
        ssum = None
        for rw_ref, rb_ref, it_ref in ((rw_l_ref, rb_l_ref, inter_l_ref),
                                       (rw_r_ref, rb_r_ref, inter_r_ref)):
            recon = lax.dot_general(f, rw_ref[...], (((1,), (1,)), ((), ())),
                                    preferred_element_type=jnp.float32)
            recon = recon + rb_ref[...]
            d = it_ref[...].astype(jnp.float32) - recon
            part = jnp.sum(d * d, axis=-1, keepdims=True)
            ssum = part if ssum is None else ssum + part
        lrow = ssum * (1.0 / span)
        lane = lax.broadcasted_iota(jnp.int32, t1.shape, 1)
        meta = jnp.where(lane == 0, lrow,
                         jnp.where(lane == 1, jnp.float32(1.0),
                                   jnp.float32(0.0)))
        tl_ref[...] = _pack_bf16_pair(t1, meta)

    @pl.when(s == zmid)
    def _zeros():
        tl_ref[...] = jnp.zeros_like(tl_ref)


def _gather_kernel(c_sref, amt_sref, tl_ref, out_ref, acc_ref, *, blk, d):
    base = pl.program_id(0) * blk
    sub = lax.broadcasted_iota(jnp.int32, (8, d), 0)
    hi_mask = jnp.full((8, d), 0xFFFF0000, jnp.uint32)
    acc = jnp.zeros((8, d), jnp.float32)
    for g8 in range(blk // 8):
        parts = []
        for j in range(8):
            i = base + g8 * 8 + j
            c = pl.multiple_of(c_sref[i], 8)
            chunk = tl_ref[pl.ds(c, 8), :]
            rolled = pltpu.roll(chunk, amt_sref[i], axis=0)
            parts.append(jnp.where(sub == j, rolled, jnp.uint32(0)))
        while len(parts) > 1:
            parts = [parts[k] | parts[k + 1] for k in range(0, len(parts), 2)]
        tile = parts[0]
        out_ref[pl.ds(g8 * 8, 8), :] = pltpu.bitcast(tile & hi_mask,
                                                     jnp.float32)
        acc = acc + pltpu.bitcast(tile << 16, jnp.float32)
    acc_ref[...] = acc


def kernel(x, emb0, emb1, inter_initial,
           ae0_w0, ae0_w1, ae0_rb0, ae0_rb1,
           ae1_w0, ae1_w1, ae1_rb0, ae1_rb1,
           rec0_w, rec0_b, rec1_w, rec1_b):
    B = x.shape[0]
    C0, K = emb0.shape
    C1 = emb1.shape[0]
    D = ae0_w1.shape[0]
    span = rec0_w.shape[0]
    h = span // 2

    TBLK = min(1024, C1)
    nb0 = C0 // TBLK
    nb1 = C1 // TBLK
    n_steps = nb0 + nb1 + 1
    n_tab = C0 + C1 + TBLK

    zmid = (n_steps - 1) // 2

    def _eff(s):
        return s - jnp.where(s > zmid, 1, 0)

    def _m0(s):
        return jnp.minimum(_eff(s) // 2, nb0 - 1)

    def _m1(s):
        return jnp.clip((_eff(s) - 1) // 2, 0, nb1 - 1)

    def _mo(s):
        e = _eff(s)
        return jnp.where(s == zmid, n_steps - 1,
                         jnp.where(e % 2 == 0, e // 2, nb0 + e // 2))

    tl = pl.pallas_call(
        functools.partial(_tables_kernel, n_steps=n_steps, span=span),
        grid=(n_steps,),
        in_specs=[
            pl.BlockSpec((TBLK, K), lambda s: (_m0(s), 0)),
            pl.BlockSpec((TBLK, K), lambda s: (_m1(s), 0)),
            pl.BlockSpec((TBLK, h), lambda s: (C0 // TBLK + _m1(s), 0)),
            pl.BlockSpec((TBLK, h), lambda s: (C0 // TBLK + _m1(s), 1)),
            pl.BlockSpec((D, K), lambda s: (0, 0)),
            pl.BlockSpec((D, D), lambda s: (0, 0)),
            pl.BlockSpec((D, K), lambda s: (0, 0)),
            pl.BlockSpec((D, D), lambda s: (0, 0)),
            pl.BlockSpec((h, D), lambda s: (0, 0)),
            pl.BlockSpec((h, D), lambda s: (1, 0)),
            pl.BlockSpec((1, h), lambda s: (0, 0)),
            pl.BlockSpec((1, h), lambda s: (0, 1)),
        ],
        out_shape=jax.ShapeDtypeStruct((n_tab, D), jnp.uint32),
        out_specs=pl.BlockSpec((TBLK, D), lambda s: (_mo(s), 0)),
        compiler_params=pltpu.CompilerParams(
            dimension_semantics=("parallel",)),
    )(emb0, emb1, inter_initial, inter_initial,
      ae0_w0, ae0_w1, ae1_w0, ae1_w1,
      rec0_w, rec0_w, rec0_b.reshape(1, span), rec0_b.reshape(1, span))

    vi = jnp.where(x == 0, C0 + C1, x - 1)
    c_arr = (vi >> 3) << 3
    amt_arr = (jnp.arange(B, dtype=jnp.int32) & 7) - (vi & 7)

    BLK = min(2048, B)
    grid2 = B // BLK
    grid_spec = pltpu.PrefetchScalarGridSpec(
        num_scalar_prefetch=2,
        grid=(grid2,),
        in_specs=[pl.BlockSpec((n_tab, D), lambda g, cs, ams: (0, 0))],
        out_specs=[pl.BlockSpec((BLK, D), lambda g, cs, ams: (g, 0)),
                   pl.BlockSpec((8, D), lambda g, cs, ams: (g, 0))],
    )
    final, accs = pl.pallas_call(
        functools.partial(_gather_kernel, blk=BLK, d=D),
        grid_spec=grid_spec,
        out_shape=(jax.ShapeDtypeStruct((B, D), jnp.float32),
                   jax.ShapeDtypeStruct((grid2 * 8, D), jnp.float32)),
        compiler_params=pltpu.CompilerParams(
            dimension_semantics=("parallel",)),
    )(c_arr, amt_arr, tl)

    lsum = jnp.sum(accs[:, 0])
    cnt = jnp.sum(accs[:, 1])
    loss = jnp.where(cnt > 0, lsum / jnp.maximum(cnt, 1.0), 0.0) * 100.0
    return final, jnp.reshape(loss, (1,))


# speedup vs baseline: 1.0128x; 1.0128x over previous
import functools

import jax
import jax.numpy as jnp
from jax import lax
from jax.experimental import pallas as pl
from jax.experimental.pallas import tpu as pltpu


def _pack_bf16_pair(hi_f32, lo_f32):
    hi = pltpu.bitcast(hi_f32.astype(jnp.bfloat16).astype(jnp.float32),
                       jnp.uint32)
    lo = pltpu.bitcast(lo_f32.astype(jnp.bfloat16).astype(jnp.float32),
                       jnp.uint32)
    return hi | (lo >> 16)


def _tables_kernel(emb0_ref, emb1_ref, inter_l_ref, inter_r_ref,
                   w00_ref, w01_ref, w10_ref, w11_ref,
                   rw_l_ref, rw_r_ref, rb_l_ref, rb_r_ref,
                   tl_ref, *, n_steps, span):
    s = pl.program_id(0)
    zmid = n_steps - 1
    eff = s - jnp.where(s > zmid, 1, 0)

    @pl.when(jnp.logical_and(s != zmid, eff % 2 == 0))
    def _t0():
        h0 = jnp.tanh(lax.dot_general(emb0_ref[...], w00_ref[...],
                                      (((1,), (1,)), ((), ())),
                                      preferred_element_type=jnp.float32))
        t0 = lax.dot_general(h0, w01_ref[...], (((1,), (1,)), ((), ())),
                             preferred_element_type=jnp.float32)
        tl_ref[...] = _pack_bf16_pair(t0, jnp.zeros_like(t0))

    @pl.when(jnp.logical_and(s != zmid, eff % 2 == 1))
    def _t1():
        h1 = jnp.tanh(lax.dot_general(emb1_ref[...], w10_ref[...],
                                      (((1,), (1,)), ((), ())),
                                      preferred_element_type=jnp.float32))
        t1 = lax.dot_general(h1, w11_ref[...], (((1,), (1,)), ((), ())),
                             preferred_element_type=jnp.float32)
        f = jnp.tanh(t1)
        ssum = None
        for rw_ref, rb_ref, it_ref in ((rw_l_ref, rb_l_ref, inter_l_ref),
                                       (rw_r_ref, rb_r_ref, inter_r_ref)):
            recon = lax.dot_general(f, rw_ref[...], (((1,), (1,)), ((), ())),
                                    preferred_element_type=jnp.float32)
            recon = recon + rb_ref[...]
            d = it_ref[...].astype(jnp.float32) - recon
            part = jnp.sum(d * d, axis=-1, keepdims=True)
            ssum = part if ssum is None else ssum + part
        lrow = ssum * (1.0 / span)
        lane = lax.broadcasted_iota(jnp.int32, t1.shape, 1)
        meta = jnp.where(lane == 0, lrow,
                         jnp.where(lane == 1, jnp.float32(1.0),
                                   jnp.float32(0.0)))
        tl_ref[...] = _pack_bf16_pair(t1, meta)

    @pl.when(s == zmid)
    def _zeros():
        tl_ref[...] = jnp.zeros_like(tl_ref)


def _gather_kernel(c_sref, amt_sref, tl_ref, out_ref, acc_ref, *, blk, d):
    base = pl.program_id(0) * blk
    sub = lax.broadcasted_iota(jnp.int32, (8, d), 0)
    hi_mask = jnp.full((8, d), 0xFFFF0000, jnp.uint32)
    acc = jnp.zeros((8, d), jnp.float32)
    for g8 in range(blk // 8):
        parts = []
        for j in range(8):
            i = base + g8 * 8 + j
            c = pl.multiple_of(c_sref[i], 8)
            chunk = tl_ref[pl.ds(c, 8), :]
            rolled = pltpu.roll(chunk, amt_sref[i], axis=0)
            parts.append(jnp.where(sub == j, rolled, jnp.uint32(0)))
        while len(parts) > 1:
            parts = [parts[k] | parts[k + 1] for k in range(0, len(parts), 2)]
        tile = parts[0]
        out_ref[pl.ds(g8 * 8, 8), :] = pltpu.bitcast(tile & hi_mask,
                                                     jnp.float32)
        acc = acc + pltpu.bitcast(tile << 16, jnp.float32)
    acc_ref[...] = acc


def kernel(x, emb0, emb1, inter_initial,
           ae0_w0, ae0_w1, ae0_rb0, ae0_rb1,
           ae1_w0, ae1_w1, ae1_rb0, ae1_rb1,
           rec0_w, rec0_b, rec1_w, rec1_b):
    B = x.shape[0]
    C0, K = emb0.shape
    C1 = emb1.shape[0]
    D = ae0_w1.shape[0]
    span = rec0_w.shape[0]
    h = span // 2

    TBLK = min(1024, C1)
    nb0 = C0 // TBLK
    nb1 = C1 // TBLK
    n_steps = nb0 + nb1 + 1
    n_tab = C0 + C1 + TBLK

    zmid = n_steps - 1

    def _eff(s):
        return s - jnp.where(s > zmid, 1, 0)

    def _m0(s):
        return jnp.minimum(_eff(s) // 2, nb0 - 1)

    def _m1(s):
        return jnp.clip((_eff(s) - 1) // 2, 0, nb1 - 1)

    def _mo(s):
        e = _eff(s)
        return jnp.where(s == zmid, n_steps - 1,
                         jnp.where(e % 2 == 0, e // 2, nb0 + e // 2))

    tl = pl.pallas_call(
        functools.partial(_tables_kernel, n_steps=n_steps, span=span),
        grid=(n_steps,),
        in_specs=[
            pl.BlockSpec((TBLK, K), lambda s: (_m0(s), 0)),
            pl.BlockSpec((TBLK, K), lambda s: (_m1(s), 0)),
            pl.BlockSpec((TBLK, h), lambda s: (C0 // TBLK + _m1(s), 0)),
            pl.BlockSpec((TBLK, h), lambda s: (C0 // TBLK + _m1(s), 1)),
            pl.BlockSpec((D, K), lambda s: (0, 0)),
            pl.BlockSpec((D, D), lambda s: (0, 0)),
            pl.BlockSpec((D, K), lambda s: (0, 0)),
            pl.BlockSpec((D, D), lambda s: (0, 0)),
            pl.BlockSpec((h, D), lambda s: (0, 0)),
            pl.BlockSpec((h, D), lambda s: (1, 0)),
            pl.BlockSpec((1, h), lambda s: (0, 0)),
            pl.BlockSpec((1, h), lambda s: (0, 1)),
        ],
        out_shape=jax.ShapeDtypeStruct((n_tab, D), jnp.uint32),
        out_specs=pl.BlockSpec((TBLK, D), lambda s: (_mo(s), 0)),
        compiler_params=pltpu.CompilerParams(
            dimension_semantics=("parallel",)),
    )(emb0, emb1, inter_initial, inter_initial,
      ae0_w0, ae0_w1, ae1_w0, ae1_w1,
      rec0_w, rec0_w, rec0_b.reshape(1, span), rec0_b.reshape(1, span))

    vi = jnp.where(x == 0, C0 + C1, x - 1)
    c_arr = (vi >> 3) << 3
    amt_arr = (jnp.arange(B, dtype=jnp.int32) & 7) - (vi & 7)

    BLK = min(2048, B)
    grid2 = B // BLK
    grid_spec = pltpu.PrefetchScalarGridSpec(
        num_scalar_prefetch=2,
        grid=(grid2,),
        in_specs=[pl.BlockSpec((n_tab, D), lambda g, cs, ams: (0, 0))],
        out_specs=[pl.BlockSpec((BLK, D), lambda g, cs, ams: (g, 0)),
                   pl.BlockSpec((8, D), lambda g, cs, ams: (g, 0))],
    )
    final, accs = pl.pallas_call(
        functools.partial(_gather_kernel, blk=BLK, d=D),
        grid_spec=grid_spec,
        out_shape=(jax.ShapeDtypeStruct((B, D), jnp.float32),
                   jax.ShapeDtypeStruct((grid2 * 8, D), jnp.float32)),
        compiler_params=pltpu.CompilerParams(
            dimension_semantics=("parallel",)),
    )(c_arr, amt_arr, tl)

    lsum = jnp.sum(accs[:, 0])
    cnt = jnp.sum(accs[:, 1])
    loss = jnp.where(cnt > 0, lsum / jnp.maximum(cnt, 1.0), 0.0) * 100.0
    return final, jnp.reshape(loss, (1,))
